# baseline (speedup 1.0000x reference)
import jax
import jax.numpy as jnp
from jax.experimental import pallas as pl

_H, _HD, _K = 4, 64, 1024
_TBLK = 256


def _vq_body(x_ref, ea_ref, e_ref, esq_ref, dist_ref, ind_ref, q_ref):
    iota = jax.lax.broadcasted_iota(jnp.int32, (_TBLK, _K), 1)
    inds = []
    for h in range(_H):
        xb = x_ref[:, h * _HD:(h + 1) * _HD]
        eb = e_ref[h]
        xb_aug = jnp.concatenate([xb, xb * xb], axis=1)
        dist = jax.lax.dot_general(
            xb_aug, ea_ref[h], (((1,), (1,)), ((), ())),
            preferred_element_type=jnp.float32)
        dist = dist - esq_ref[h:h + 1, :]
        dist_ref[:, h, :] = dist

        m = jnp.max(dist, axis=1, keepdims=True)
        ind = jnp.min(jnp.where(dist == m, iota, _K), axis=1, keepdims=True)
        inds.append(ind)

        onehot = (iota == ind).astype(jnp.float32)
        q = jax.lax.dot_general(
            onehot, eb, (((1,), (0,)), ((), ())),
            preferred_element_type=jnp.float32)
        q_ref[:, h * _HD:(h + 1) * _HD] = q

    ind_ref[...] = jnp.concatenate(inds, axis=1)


@jax.jit
def kernel(x, x_len, embed):
    B, T, D = x.shape
    BT = B * T
    xf = x.reshape(BT, D)
    n_t = BT // _TBLK
    e_sq = jnp.sum(embed * embed, axis=-1)
    e_aug = jnp.concatenate(
        [embed * 2.0, -jnp.ones_like(embed)], axis=-1)

    dist, ind, quant = pl.pallas_call(
        _vq_body,
        grid=(n_t,),
        in_specs=[
            pl.BlockSpec((_TBLK, D), lambda i: (i, 0)),
            pl.BlockSpec((_H, _K, 2 * _HD), lambda i: (0, 0, 0)),
            pl.BlockSpec((_H, _K, _HD), lambda i: (0, 0, 0)),
            pl.BlockSpec((_H, _K), lambda i: (0, 0)),
        ],
        out_specs=[
            pl.BlockSpec((_TBLK, _H, _K), lambda i: (i, 0, 0)),
            pl.BlockSpec((_TBLK, _H), lambda i: (i, 0)),
            pl.BlockSpec((_TBLK, D), lambda i: (i, 0)),
        ],
        out_shape=[
            jax.ShapeDtypeStruct((BT, _H, _K), jnp.float32),
            jax.ShapeDtypeStruct((BT, _H), jnp.int32),
            jax.ShapeDtypeStruct((BT, D), jnp.float32),
        ],
    )(xf, e_aug, embed, e_sq)

    return (quant.reshape(B, T, D),
            ind.reshape(B, T, _H),
            dist.reshape(B, T, _H, _K))


# speedup vs baseline: 1.0681x; 1.0681x over previous
import jax
import jax.numpy as jnp
from jax.experimental import pallas as pl

_H, _HD, _K = 4, 64, 1024
_TBLK = 1024


def _vq_body(x_ref, ea_ref, e_ref, esq_ref, dist_ref, ind_ref, q_ref):
    iota = jax.lax.broadcasted_iota(jnp.int32, (_TBLK, _K), 1)
    inds = []
    for h in range(_H):
        xb = x_ref[:, h * _HD:(h + 1) * _HD]
        eb = e_ref[h]
        xb_aug = jnp.concatenate([xb, xb * xb], axis=1)
        dist = jax.lax.dot_general(
            xb_aug, ea_ref[h], (((1,), (1,)), ((), ())),
            preferred_element_type=jnp.float32)
        dist = dist - esq_ref[h:h + 1, :]
        dist_ref[:, h, :] = dist

        m = jnp.max(dist, axis=1, keepdims=True)
        ind = jnp.min(jnp.where(dist == m, iota, _K), axis=1, keepdims=True)
        inds.append(ind)

        onehot = (iota == ind).astype(jnp.float32)
        q = jax.lax.dot_general(
            onehot, eb, (((1,), (0,)), ((), ())),
            preferred_element_type=jnp.float32)
        q_ref[:, h * _HD:(h + 1) * _HD] = q

    ind_ref[...] = jnp.concatenate(inds, axis=1)


@jax.jit
def kernel(x, x_len, embed):
    B, T, D = x.shape
    BT = B * T
    xf = x.reshape(BT, D)
    n_t = BT // _TBLK
    e_sq = jnp.sum(embed * embed, axis=-1)
    e_aug = jnp.concatenate(
        [embed * 2.0, -jnp.ones_like(embed)], axis=-1)

    dist, ind, quant = pl.pallas_call(
        _vq_body,
        grid=(n_t,),
        in_specs=[
            pl.BlockSpec((_TBLK, D), lambda i: (i, 0)),
            pl.BlockSpec((_H, _K, 2 * _HD), lambda i: (0, 0, 0)),
            pl.BlockSpec((_H, _K, _HD), lambda i: (0, 0, 0)),
            pl.BlockSpec((_H, _K), lambda i: (0, 0)),
        ],
        out_specs=[
            pl.BlockSpec((_TBLK, _H, _K), lambda i: (i, 0, 0)),
            pl.BlockSpec((_TBLK, _H), lambda i: (i, 0)),
            pl.BlockSpec((_TBLK, D), lambda i: (i, 0)),
        ],
        out_shape=[
            jax.ShapeDtypeStruct((BT, _H, _K), jnp.float32),
            jax.ShapeDtypeStruct((BT, _H), jnp.int32),
            jax.ShapeDtypeStruct((BT, D), jnp.float32),
        ],
    )(xf, e_aug, embed, e_sq)

    return (quant.reshape(B, T, D),
            ind.reshape(B, T, _H),
            dist.reshape(B, T, _H, _K))
